# baseline (speedup 1.0000x reference)
import jax, jax.numpy as jnp
from jax.experimental import pallas as pl
from jax.experimental.pallas import tpu as pltpu

def _body(s_ref, cpf_ref, xf_ref, nf_ref, op_ref, ox_ref):
    op_ref[...] = cpf_ref[...] * s_ref[1]
    ox_ref[...] = xf_ref[...] + nf_ref[...] * s_ref[0]

def kernel(x, h, composition_probs, num_atoms, t):
    N, C = x.shape
    A = composition_probs.shape[1]
    sigmas = jnp.exp(jnp.linspace(jnp.log(10.0), jnp.log(0.01), 50)).astype(jnp.float32)
    type_sigmas = jnp.exp(jnp.linspace(jnp.log(5.0), jnp.log(0.01), 50)).astype(jnp.float32)
    sv = jnp.stack([sigmas[t], type_sigmas[t]])
    nkey = jax.random.fold_in(jax.random.key(0), 1234)
    noise = jax.random.normal(nkey, x.shape, x.dtype)
    cpf = composition_probs.reshape(N*A//128, 128)
    xf = x.reshape(N*C//128, 128)
    nf = noise.reshape(N*C//128, 128)
    G = 64
    RC = cpf.shape[0] // G
    RX = xf.shape[0] // G
    op, ox = pl.pallas_call(
        _body,
        grid=(G,),
        in_specs=[
            pl.BlockSpec(memory_space=pltpu.SMEM),
            pl.BlockSpec((RC, 128), lambda i: (i, 0)),
            pl.BlockSpec((RX, 128), lambda i: (i, 0)),
            pl.BlockSpec((RX, 128), lambda i: (i, 0)),
        ],
        out_specs=[
            pl.BlockSpec((RC, 128), lambda i: (i, 0)),
            pl.BlockSpec((RX, 128), lambda i: (i, 0)),
        ],
        out_shape=[
            jax.ShapeDtypeStruct(cpf.shape, jnp.float32),
            jax.ShapeDtypeStruct(xf.shape, jnp.float32),
        ],
    )(sv, cpf, xf, nf)
    return (ox.reshape(N, C), op.reshape(N, A))


# speedup vs baseline: 4.1250x; 4.1250x over previous
import functools

import jax
import jax.numpy as jnp
from jax import lax
from jax.experimental import pallas as pl
from jax.experimental.pallas import tpu as pltpu

_NUM = 50


def _ring_body(sig_ref, t_ref, comp_hbm, h_hbm, op_hbm,
               in_bufs, out_bufs, h_vmem, in_sems, out_sems, h_sem,
               *, CH, K, NCH):
    tt = t_ref[0]
    ts = sig_ref[_NUM + tt]
    rows_per_tile = CH // 128

    hd = pltpu.make_async_copy(h_hbm, h_vmem, h_sem)
    hd.start()
    hd.wait()

    def in_dma(c, slot):
        return pltpu.make_async_copy(
            comp_hbm.at[pl.ds(c * CH, CH), :], in_bufs.at[slot],
            in_sems.at[slot])

    def out_dma(c, slot):
        return pltpu.make_async_copy(
            out_bufs.at[slot], op_hbm.at[pl.ds(c * CH, CH), :],
            out_sems.at[slot])

    for k in range(K - 1):
        in_dma(k, k).start()

    def step(c, _):
        slot = lax.rem(c, K)
        nxt = c + K - 1

        @pl.when(nxt < NCH)
        def _():
            in_dma(nxt, lax.rem(nxt, K)).start()

        @pl.when(c >= K)
        def _():
            out_dma(c - K, slot).wait()

        in_dma(c, slot).wait()

        hh = h_vmem[pl.ds(c * rows_per_tile, rows_per_tile), :] - 1
        hh_t = jnp.transpose(hh)
        cols = [
            lax.slice(hh_t, (0, q), (128, q + 1))
            for q in range(rows_per_tile)
        ]
        hm1_col = jnp.concatenate(cols, axis=0)
        lanes = lax.broadcasted_iota(jnp.int32, (CH, 100), 1)
        onehot = (lanes == hm1_col).astype(jnp.float32)
        out_bufs[slot] = in_bufs[slot] * ts + onehot
        out_dma(c, slot).start()
        return 0

    lax.fori_loop(0, NCH, step, 0)

    def drain(c, _):
        out_dma(c, lax.rem(c, K)).wait()
        return 0

    lax.fori_loop(NCH - K, NCH, drain, 0)


def kernel(x, h, composition_probs, num_atoms, t):
    N, C = x.shape
    A = composition_probs.shape[1]
    assert A == 100

    sigmas = jnp.exp(jnp.linspace(jnp.log(10.0), jnp.log(0.01), _NUM)).astype(jnp.float32)
    type_sigmas = jnp.exp(jnp.linspace(jnp.log(5.0), jnp.log(0.01), _NUM)).astype(jnp.float32)
    sig_all = jnp.concatenate([sigmas, type_sigmas])
    t_arr = jnp.asarray(t, dtype=jnp.int32).reshape(1)

    CH = 8192
    K = 4
    NCH = N // CH
    h2 = h.reshape(N // 128, 128)

    op = pl.pallas_call(
        functools.partial(_ring_body, CH=CH, K=K, NCH=NCH),
        in_specs=[
            pl.BlockSpec(memory_space=pltpu.SMEM),
            pl.BlockSpec(memory_space=pltpu.SMEM),
            pl.BlockSpec(memory_space=pl.ANY),
            pl.BlockSpec(memory_space=pl.ANY),
        ],
        out_specs=pl.BlockSpec(memory_space=pl.ANY),
        out_shape=jax.ShapeDtypeStruct((N, A), jnp.float32),
        scratch_shapes=[
            pltpu.VMEM((K, CH, A), jnp.float32),
            pltpu.VMEM((K, CH, A), jnp.float32),
            pltpu.VMEM((N // 128, 128), jnp.int32),
            pltpu.SemaphoreType.DMA((K,)),
            pltpu.SemaphoreType.DMA((K,)),
            pltpu.SemaphoreType.DMA,
        ],
    )(sig_all, t_arr, composition_probs, h2)

    nkey = jax.random.fold_in(jax.random.key(0), 1234)
    noise = jax.random.normal(nkey, x.shape, x.dtype)
    out_x = x + noise * sigmas[t]
    return (out_x, op)
